# baseline (speedup 1.0000x reference)
import functools
import math

import jax
import jax.numpy as jnp
from jax import lax
from jax.experimental import pallas as pl
from jax.experimental.pallas import tpu as pltpu
from jax.experimental.pallas import tpu_sc as plsc

_IN_F = 4096
_OUT_F = 64
_K = 8
_NEG_INF = float("-inf")
_N_WORKERS = 32
_LANES = 16


_STRIDE = 65


def _matmul_body(x_ref, wt_ref, b_ref, out_ref):
    logits = jax.lax.dot_general(
        x_ref[...], wt_ref[...], (((1,), (0,)), ((), ())),
        preferred_element_type=jnp.float32,
    ) + b_ref[...]
    pad = jnp.zeros((logits.shape[0], _STRIDE - _OUT_F), jnp.float32)
    out_ref[...] = jnp.concatenate([logits, pad], axis=1)


@functools.partial(jax.jit, static_argnames=("block",))
def _logits_tc(x, wt, b2d, block=1024):
    n = x.shape[0]
    return pl.pallas_call(
        _matmul_body,
        grid=(n // block,),
        in_specs=[
            pl.BlockSpec((block, _IN_F), lambda i: (i, 0)),
            pl.BlockSpec((_IN_F, _OUT_F), lambda i: (0, 0)),
            pl.BlockSpec((1, _OUT_F), lambda i: (0, 0)),
        ],
        out_specs=pl.BlockSpec((block, _STRIDE), lambda i: (i, 0)),
        out_shape=jax.ShapeDtypeStruct((n, _STRIDE), jnp.float32),
        compiler_params=pltpu.CompilerParams(
            dimension_semantics=("arbitrary",),
        ),
    )(x, wt, b2d)


def _topk_sc_body(logits_hbm, vals_hbm, idx_hbm, buf, vals_v, idx_v):
    n_tok = logits_hbm.shape[0] // _STRIDE
    t_per_w = n_tok // _N_WORKERS
    n_groups = t_per_w // _LANES
    wid = lax.axis_index("c") * 16 + lax.axis_index("s")
    base = wid * t_per_w
    pltpu.sync_copy(logits_hbm.at[pl.ds(base * _STRIDE, t_per_w * _STRIDE)], buf)

    lane = lax.broadcasted_iota(jnp.int32, (_LANES,), 0)
    neg_inf_v = jnp.full((_LANES,), _NEG_INF, jnp.float32)

    def _tree(vs, is_):
        while len(vs) > 1:
            nvs, nis = [], []
            for j in range(0, len(vs), 2):
                keep = vs[j] >= vs[j + 1]
                nvs.append(jnp.where(keep, vs[j], vs[j + 1]))
                nis.append(jnp.where(keep, is_[j], is_[j + 1]))
            vs, is_ = nvs, nis
        return vs[0], is_[0]

    n_chunks = _OUT_F // _LANES

    def group_body(g, carry):
        row = (g * _LANES + lane) * _STRIDE
        out_row = (g * _LANES + lane) * _K

        cvs, cis = [], []
        for c in range(n_chunks):
            vs = [plsc.load_gather(buf, [row + (c * _LANES + j)])
                  for j in range(_LANES)]
            is_ = [jnp.full((_LANES,), c * _LANES + j, jnp.int32)
                   for j in range(_LANES)]
            cv, ci = _tree(vs, is_)
            cvs.append(cv)
            cis.append(ci)

        def pass_body(k, carry2):
            cv_l = list(carry2[:n_chunks])
            ci_l = list(carry2[n_chunks:])
            m, mi = _tree(list(cv_l), list(ci_l))
            plsc.store_scatter(vals_v, [out_row + k], m)
            plsc.store_scatter(idx_v, [out_row + k], mi)
            plsc.store_scatter(buf, [row + mi], neg_inf_v)
            cb = jnp.bitwise_and(mi, jnp.full((_LANES,), -_LANES, jnp.int32))
            vs = [plsc.load_gather(buf, [row + cb + j]) for j in range(_LANES)]
            is_ = [cb + j for j in range(_LANES)]
            nv, ni = _tree(vs, is_)
            cid = jnp.right_shift(mi, 4)
            for c in range(n_chunks):
                hit = cid == c
                cv_l[c] = jnp.where(hit, nv, cv_l[c])
                ci_l[c] = jnp.where(hit, ni, ci_l[c])
            return tuple(cv_l) + tuple(ci_l)

        lax.fori_loop(0, _K, pass_body, tuple(cvs) + tuple(cis))
        return carry

    lax.fori_loop(0, n_groups, group_body, 0)
    pltpu.sync_copy(vals_v, vals_hbm.at[pl.ds(base * _K, t_per_w * _K)])
    pltpu.sync_copy(idx_v, idx_hbm.at[pl.ds(base * _K, t_per_w * _K)])


@jax.jit
def _topk_sc(logits):
    n = logits.shape[0]
    t_per_w = n // _N_WORKERS
    mesh = plsc.VectorSubcoreMesh(core_axis_name="c", subcore_axis_name="s")
    f = functools.partial(
        pl.kernel,
        out_type=[
            jax.ShapeDtypeStruct((n * _K,), jnp.float32),
            jax.ShapeDtypeStruct((n * _K,), jnp.int32),
        ],
        mesh=mesh,
        scratch_types=[
            pltpu.VMEM((t_per_w * _STRIDE,), jnp.float32),
            pltpu.VMEM((t_per_w * _K,), jnp.float32),
            pltpu.VMEM((t_per_w * _K,), jnp.int32),
        ],
        compiler_params=pltpu.CompilerParams(needs_layout_passes=False),
    )(_topk_sc_body)
    vals, idx = f(logits.reshape(-1))
    return vals.reshape(n, _K), idx.reshape(n, _K)


def kernel(input, weight, bias):
    wt = weight.T
    b2d = bias.reshape(1, _OUT_F)
    n = input.shape[0]
    n_chunks = 4
    c = n // n_chunks
    vs, is_ = [], []
    for i in range(n_chunks):
        logits = _logits_tc(input[i * c:(i + 1) * c], wt, b2d)
        v, ix = _topk_sc(logits)
        vs.append(v)
        is_.append(ix)
    return (jnp.concatenate(vs), jnp.concatenate(is_))


# speedup vs baseline: 2.0221x; 2.0221x over previous
import functools
import math

import jax
import jax.numpy as jnp
from jax.experimental import pallas as pl
from jax.experimental.pallas import tpu as pltpu

_IN_F = 4096
_OUT_F = 64
_K = 8
_NEG_INF = float("-inf")


def _topk_write(logits, vals_ref, idx_ref):
    col = jax.lax.broadcasted_iota(jnp.int32, logits.shape, 1)
    alive = col < _OUT_F
    vals_cols = []
    idx_cols = []
    for _ in range(_K):
        masked = jnp.where(alive, logits, _NEG_INF)
        m = jnp.max(masked, axis=1, keepdims=True)
        hit = jnp.logical_and(alive, masked == m)
        pick = jnp.min(jnp.where(hit, col, _OUT_F), axis=1, keepdims=True)
        vals_cols.append(m)
        idx_cols.append(pick)
        alive = jnp.logical_and(alive, col != pick)
    vals_ref[...] = jnp.concatenate(vals_cols, axis=1)
    idx_ref[...] = jnp.concatenate(idx_cols, axis=1)


def _fused_body(x_ref, wt_ref, b_ref, vals_ref, idx_ref, acc_ref, *, nj):
    j = pl.program_id(1)
    part = jax.lax.dot_general(
        x_ref[...], wt_ref[...], (((1,), (0,)), ((), ())),
        preferred_element_type=jnp.float32,
    )

    @pl.when(j == 0)
    def _init():
        acc_ref[...] = part + b_ref[...]

    @pl.when(j != 0)
    def _acc():
        acc_ref[...] += part

    @pl.when(j == nj - 1)
    def _emit():
        _topk_write(acc_ref[...], vals_ref, idx_ref)


@functools.partial(jax.jit, static_argnames=("block", "fblk"))
def _run(x, wt, b2d, block=2048, fblk=2048):
    n = x.shape[0]
    nj = _IN_F // fblk
    return pl.pallas_call(
        functools.partial(_fused_body, nj=nj),
        grid=(n // block, nj),
        in_specs=[
            pl.BlockSpec((block, fblk), lambda i, j: (i, j)),
            pl.BlockSpec((fblk, _OUT_F), lambda i, j: (j, 0)),
            pl.BlockSpec((1, _OUT_F), lambda i, j: (0, 0)),
        ],
        out_specs=[
            pl.BlockSpec((block, _K), lambda i, j: (i, 0)),
            pl.BlockSpec((block, _K), lambda i, j: (i, 0)),
        ],
        out_shape=[
            jax.ShapeDtypeStruct((n, _K), jnp.float32),
            jax.ShapeDtypeStruct((n, _K), jnp.int32),
        ],
        scratch_shapes=[pltpu.VMEM((block, _OUT_F), jnp.float32)],
        compiler_params=pltpu.CompilerParams(
            dimension_semantics=("arbitrary", "arbitrary"),
        ),
    )(x, wt, b2d)


def kernel(input, weight, bias):
    wt = weight.T
    b2d = bias.reshape(1, _OUT_F)
    vals, idx = _run(input, wt, b2d)
    return (vals, idx)


# speedup vs baseline: 2.7468x; 1.3584x over previous
import functools
import math

import jax
import jax.numpy as jnp
from jax.experimental import pallas as pl
from jax.experimental.pallas import tpu as pltpu

_IN_F = 4096
_OUT_F = 64
_K = 8
_NEG_INF = float("-inf")


def _fused_body(x_ref, wt_ref, b_ref, vals_ref, idx_ref):
    x = x_ref[...]
    wt = wt_ref[...]
    logits = jax.lax.dot_general(
        x, wt, (((1,), (0,)), ((), ())),
        preferred_element_type=jnp.float32,
    ) + b_ref[...]

    col = jax.lax.broadcasted_iota(jnp.int32, logits.shape, 1)
    vals_cols = []
    idx_cols = []
    for _ in range(_K):
        m = jnp.max(logits, axis=1, keepdims=True)
        pick = jnp.min(
            jnp.where(logits == m, col, _OUT_F), axis=1, keepdims=True
        )
        vals_cols.append(m)
        idx_cols.append(pick)
        logits = jnp.where(col == pick, _NEG_INF, logits)
    vals_ref[...] = jnp.concatenate(vals_cols, axis=1)
    idx_ref[...] = jnp.concatenate(idx_cols, axis=1)


@functools.partial(jax.jit, static_argnames=("block",))
def _run(x, wt, b2d, block=1024):
    n = x.shape[0]
    grid = (n // block,)
    return pl.pallas_call(
        _fused_body,
        grid=grid,
        in_specs=[
            pl.BlockSpec((block, _IN_F), lambda i: (i, 0)),
            pl.BlockSpec((_IN_F, _OUT_F), lambda i: (0, 0)),
            pl.BlockSpec((1, _OUT_F), lambda i: (0, 0)),
        ],
        out_specs=[
            pl.BlockSpec((block, _K), lambda i: (i, 0)),
            pl.BlockSpec((block, _K), lambda i: (i, 0)),
        ],
        out_shape=[
            jax.ShapeDtypeStruct((n, _K), jnp.float32),
            jax.ShapeDtypeStruct((n, _K), jnp.int32),
        ],
        compiler_params=pltpu.CompilerParams(
            dimension_semantics=("arbitrary",),
        ),
    )(x, wt, b2d)


def kernel(input, weight, bias):
    wt = weight.T
    b2d = bias.reshape(1, _OUT_F)
    vals, idx = _run(input, wt, b2d)
    return (vals, idx)
